# Initial kernel scaffold; baseline (speedup 1.0000x reference)
import jax
import jax.numpy as jnp
from jax.experimental import pallas as pl


def kernel(node_fts, edge_fts, edges, W_node, W_edge, attn_a):
    raise NotImplementedError("write your pallas kernel here")


# speedup vs baseline: 5.8478x; 5.8478x over previous
import jax
import jax.numpy as jnp
from jax import lax
from jax.experimental import pallas as pl
from jax.experimental.pallas import tpu as pltpu
from jax.experimental.pallas import tpu_sc as plsc

N = 10000
E = 320000
D_IN = 128
D_OUT = 128
E_IN = 16
E_OUT = 16
H = 4
ALPHA = 0.2

NC = 2
NS = 16
L = 16

NPAD = 10240
PERT = NPAD // NS
EPT = E // NS
CH = 80
NCHUNK = EPT // CH
JV = CH // L
CW = 512
SUB = CW // NS
QP = 4
DQ = D_OUT // QP


def _iota16():
  return lax.iota(jnp.int32, L)


def _shift_left(v, k):
  idx = jnp.minimum(_iota16() + k, L - 1)
  return jnp.take_along_axis(v, idx, axis=0, mode="promise_in_bounds")


def _shift_right1(v):
  idx = jnp.maximum(_iota16() - 1, 0)
  return jnp.take_along_axis(v, idx, axis=0, mode="promise_in_bounds")


def _run_masks(dsts):
  it = _iota16()
  first = (it == 0) | (dsts != _shift_right1(dsts))
  last = (it == L - 1) | (dsts != _shift_left(dsts, 1))
  return first, last


def _sc_kernel_body(src_hbm, dst_hbm, hq, e_tab, s0q, as_tab, ad_tab,
                    ohq, oeq, stats,
                    srcb_v, dst_t, sx_t,
                    asrc_v, adst_v, acc_v, comb_v,
                    cstage_v, redtmp_v, rowb_v, erow_v, stat_v, statall_v,
                    outh_s, oute_s, comb_s, red_s, stats_s,
                    sem, sem2):
  sid = lax.axis_index("s")
  cid = lax.axis_index("c")

  zero16 = jnp.zeros((L,), jnp.float32)

  pltpu.sync_copy(dst_hbm.at[sid], dst_t)

  def src_wait(i):
    slot = lax.rem(i, 2)
    pltpu.make_async_copy(src_hbm.at[sid, i], srcb_v.at[slot], sem2).wait()

    @pl.when(i + 1 < NCHUNK)
    def _():
      pltpu.async_copy(src_hbm.at[sid, i + 1],
                       srcb_v.at[lax.rem(i + 1, 2)], sem2)
    return slot

  def src_prime():
    pltpu.async_copy(src_hbm.at[sid, 0], srcb_v.at[0], sem2)

  def zero_rowb(r, _):
    for k in range(DQ // L):
      rowb_v[r, pl.ds(k * L, L)] = zero16
    erow_v[r, :] = zero16
    return 0

  def zero_outh(j, _):
    pltpu.sync_copy(rowb_v, outh_s.at[pl.ds(sid * PERT + j * CH, CH)])
    return 0

  def process_head(t, _):
    hh = cid * 2 + t
    pltpu.sync_copy(s0q.at[hh, sid], sx_t)
    pltpu.sync_copy(as_tab.at[hh], asrc_v)
    pltpu.sync_copy(ad_tab.at[hh], adst_v)

    lax.fori_loop(0, CH, zero_rowb, 0)

    def zero_acc_copy(j, _):
      base = sid * PERT + j * CH
      pltpu.sync_copy(rowb_v, outh_s.at[pl.ds(base, CH)])
      pltpu.sync_copy(erow_v, oute_s.at[pl.ds(base, CH)])
      return 0
    lax.fori_loop(0, PERT // CH, zero_acc_copy, 0)

    def init_acc(j, val):
      acc_v[pl.ds(j * L, L)] = val
      return val
    lax.fori_loop(0, NPAD // L, init_acc, jnp.full((L,), -3e38, jnp.float32))

    src_prime()

    def phase_a(i, _):
      slot = src_wait(i)
      for j in range(JV):
        off = pl.ds(j * L, L)
        src16 = srcb_v[slot, off]
        dst16 = dst_t[i, off]
        s = (plsc.load_gather(asrc_v, [src16])
             + plsc.load_gather(adst_v, [dst16])
             + sx_t[i, off])
        s = jnp.where(s > 0, s, ALPHA * s)
        sx_t[i, off] = s
        dsts, ss = plsc.sort_key_val(dst16, s)
        m = ss
        for k in (1, 2, 4, 8):
          idx = jnp.minimum(_iota16() + k, L - 1)
          cand = jnp.take_along_axis(m, idx, axis=0, mode="promise_in_bounds")
          same = dsts == jnp.take_along_axis(dsts, idx, axis=0,
                                             mode="promise_in_bounds")
          m = jnp.where(same, jnp.maximum(m, cand), m)
        first, _ = _run_masks(dsts)
        cur = plsc.load_gather(acc_v, [dsts])
        plsc.store_scatter(acc_v, [dsts], jnp.maximum(cur, m), mask=first)
      return 0
    lax.fori_loop(0, NCHUNK, phase_a, 0)

    def repair_sweep(i, ch):
      for j in range(JV):
        off = pl.ds(j * L, L)
        dst16 = dst_t[i, off]
        s = sx_t[i, off]
        cur = plsc.load_gather(acc_v, [dst16])
        need = s > cur
        plsc.store_scatter(acc_v, [dst16], jnp.maximum(cur, s), mask=need)
        ch = ch | jnp.any(need)
      return ch

    lax.while_loop(
        lambda c: c,
        lambda c: lax.fori_loop(0, NCHUNK, repair_sweep, jnp.bool_(False)),
        jnp.bool_(True))

    def combine(op_is_max):
      plsc.subcore_barrier()

      def round_body(r, _):
        pltpu.sync_copy(acc_v.at[pl.ds(r * CW, CW)],
                        comb_s.at[pl.ds(sid * CW, CW)])
        plsc.subcore_barrier()
        for t in range(NS):
          pltpu.sync_copy(comb_s.at[pl.ds(t * CW + sid * SUB, SUB)],
                          cstage_v.at[pl.ds(t * SUB, SUB)])
        for j in range(SUB // L):
          v = cstage_v[pl.ds(j * L, L)]
          for t in range(1, NS):
            w = cstage_v[pl.ds(t * SUB + j * L, L)]
            v = jnp.maximum(v, w) if op_is_max else v + w
          redtmp_v[pl.ds(j * L, L)] = v
        pltpu.sync_copy(redtmp_v, red_s.at[pl.ds(r * CW + sid * SUB, SUB)])
        plsc.subcore_barrier()
        return 0
      lax.fori_loop(0, NPAD // CW, round_body, 0)
      pltpu.sync_copy(red_s, comb_v)

    combine(op_is_max=True)

    lax.fori_loop(0, NPAD // L, init_acc, jnp.zeros((L,), jnp.float32))

    def phase_b(i, _):
      for j in range(JV):
        off = pl.ds(j * L, L)
        dst16 = dst_t[i, off]
        m16 = plsc.load_gather(comb_v, [dst16])
        ex = jnp.exp(jnp.minimum(sx_t[i, off] - m16, 80.0))
        sx_t[i, off] = ex
        dsts, exs = plsc.sort_key_val(dst16, ex)
        c = plsc.cumsum(exs)
        prev = jnp.where(_iota16() == 0, 0.0, _shift_right1(c))
        first, last = _run_masks(dsts)
        base = plsc.cummax(jnp.where(first, prev, 0.0))
        run_sum = c - base
        plsc.addupdate_scatter(acc_v, [dsts], run_sum, mask=last)
      return 0
    lax.fori_loop(0, NCHUNK, phase_b, 0)

    combine(op_is_max=False)

    src_prime()

    def phase_c0(i, carry):
      sa, sa2 = carry
      slot = src_wait(i)
      for j in range(JV):
        off = pl.ds(j * L, L)
        dst16 = dst_t[i, off]
        den16 = plsc.load_gather(comb_v, [dst16])
        a = sx_t[i, off] / (den16 + 1e-9)
        sx_t[i, off] = a
        sa = sa + a
        sa2 = sa2 + a * a
      pltpu.async_copy(hq.at[hh * QP].at[srcb_v.at[slot]], rowb_v, sem).wait()
      eoff = sid * EPT + i * CH
      pltpu.sync_copy(e_tab.at[hh, pl.ds(eoff, CH)], erow_v)

      ri = lax.broadcast(i, (L,))

      def scale_row(r, _):
        ar = plsc.load_gather(sx_t, [ri, lax.broadcast(r, (L,))])
        for k in range(DQ // L):
          o = pl.ds(k * L, L)
          rowb_v[r, o] = rowb_v[r, o] * ar
        erow_v[r, :] = erow_v[r, :] * ar
        return 0
      lax.fori_loop(0, CH, scale_row, 0)

      pltpu.sync_copy(rowb_v, outh_s.at[dst_t.at[i]], add=True)
      pltpu.sync_copy(erow_v, oute_s.at[dst_t.at[i]], add=True)
      return (sa, sa2)

    sa, sa2 = lax.fori_loop(0, NCHUNK, phase_c0,
                            (jnp.zeros((L,), jnp.float32),
                             jnp.zeros((L,), jnp.float32)))

    it = _iota16()
    stat_v[:] = jnp.where(it == 0, jnp.sum(sa),
                          jnp.where(it == 1, jnp.sum(sa2), 0.0))
    pltpu.sync_copy(stat_v, stats_s.at[pl.ds(sid * L, L)])
    plsc.subcore_barrier()

    @pl.when(sid == 0)
    def _():
      pltpu.sync_copy(stats_s, statall_v)
      r = statall_v[pl.ds(0, L)]
      for t in range(1, NS):
        r = r + statall_v[pl.ds(t * L, L)]
      stat_v[:] = r
      pltpu.sync_copy(stat_v, stats.at[hh])

    base = sid * PERT

    pltpu.sync_copy(oute_s.at[pl.ds(base, PERT)],
                    oeq.at[hh, pl.ds(base, PERT)])

    def h_pass(q, _):
      plsc.subcore_barrier()
      pltpu.sync_copy(outh_s.at[pl.ds(base, PERT)],
                      ohq.at[hh * QP + q - 1, pl.ds(base, PERT)])
      lax.fori_loop(0, CH, zero_rowb, 0)
      lax.fori_loop(0, PERT // CH, zero_outh, 0)
      plsc.subcore_barrier()

      src_prime()

      def body(i, _):
        slot = src_wait(i)
        pltpu.async_copy(hq.at[hh * QP + q].at[srcb_v.at[slot]], rowb_v,
                         sem).wait()
        ri = lax.broadcast(i, (L,))

        def scale_row1(r, _):
          ar = plsc.load_gather(sx_t, [ri, lax.broadcast(r, (L,))])
          for k in range(DQ // L):
            o = pl.ds(k * L, L)
            rowb_v[r, o] = rowb_v[r, o] * ar
          return 0
        lax.fori_loop(0, CH, scale_row1, 0)
        pltpu.sync_copy(rowb_v, outh_s.at[dst_t.at[i]], add=True)
        return 0
      lax.fori_loop(0, NCHUNK, body, 0)
      return 0
    lax.fori_loop(1, QP, h_pass, 0)

    plsc.subcore_barrier()
    pltpu.sync_copy(outh_s.at[pl.ds(base, PERT)],
                    ohq.at[hh * QP + QP - 1, pl.ds(base, PERT)])
    plsc.subcore_barrier()
    return 0

  lax.fori_loop(0, H // NC, process_head, 0)


def _make_sc_call():
  mesh = plsc.VectorSubcoreMesh(core_axis_name="c", subcore_axis_name="s",
                                num_cores=NC, num_subcores=NS)
  out_type = (
      jax.ShapeDtypeStruct((H * QP, NPAD, DQ), jnp.float32),
      jax.ShapeDtypeStruct((H, NPAD, E_OUT), jnp.float32),
      jax.ShapeDtypeStruct((H, L), jnp.float32),
  )
  scratch = [
      pltpu.VMEM((2, CH), jnp.int32),
      pltpu.VMEM((NCHUNK, CH), jnp.int32),
      pltpu.VMEM((NCHUNK, CH), jnp.float32),
      pltpu.VMEM((NPAD,), jnp.float32),
      pltpu.VMEM((NPAD,), jnp.float32),
      pltpu.VMEM((NPAD,), jnp.float32),
      pltpu.VMEM((NPAD,), jnp.float32),
      pltpu.VMEM((NS * SUB,), jnp.float32),
      pltpu.VMEM((SUB,), jnp.float32),
      pltpu.VMEM((CH, DQ), jnp.float32),
      pltpu.VMEM((CH, E_OUT), jnp.float32),
      pltpu.VMEM((L,), jnp.float32),
      pltpu.VMEM((NS * L,), jnp.float32),
      pltpu.VMEM_SHARED((NPAD, DQ), jnp.float32),
      pltpu.VMEM_SHARED((NPAD, E_OUT), jnp.float32),
      pltpu.VMEM_SHARED((NS * CW,), jnp.float32),
      pltpu.VMEM_SHARED((NPAD,), jnp.float32),
      pltpu.VMEM_SHARED((NS * L,), jnp.float32),
      pltpu.SemaphoreType.DMA,
      pltpu.SemaphoreType.DMA,
  ]
  return pl.kernel(_sc_kernel_body, out_type=out_type, mesh=mesh,
                   scratch_types=scratch,
                   compiler_params=pltpu.CompilerParams(
                       needs_layout_passes=False,
                       use_tc_tiling_on_sc=False))


def _tc_node_proj(node_fts, wn_all, a_alpha):
  nb = 1000
  grid = (N // nb,)

  def body(x_ref, w_ref, a_ref, *out_refs):
    h = jnp.dot(x_ref[...], w_ref[...], preferred_element_type=jnp.float32)
    for t in range(H):
      for q in range(QP):
        c0 = t * D_OUT + q * DQ
        out_refs[t * QP + q][...] = h[:, c0:c0 + DQ]
    out_refs[H * QP][...] = jnp.dot(h, a_ref[...],
                                    preferred_element_type=jnp.float32)

  return pl.pallas_call(
      body,
      grid=grid,
      in_specs=[
          pl.BlockSpec((nb, D_IN), lambda i: (i, 0)),
          pl.BlockSpec((D_IN, H * D_OUT), lambda i: (0, 0)),
          pl.BlockSpec((H * D_OUT, 2 * H), lambda i: (0, 0)),
      ],
      out_specs=[pl.BlockSpec((nb, DQ), lambda i: (i, 0))
                 for _ in range(H * QP)]
      + [pl.BlockSpec((nb, 2 * H), lambda i: (i, 0))],
      out_shape=[jax.ShapeDtypeStruct((N, DQ), jnp.float32)
                 for _ in range(H * QP)]
      + [jax.ShapeDtypeStruct((N, 2 * H), jnp.float32)],
  )(node_fts, wn_all, a_alpha)


def _tc_edge_proj(edge_fts, we_all, a_e):
  eb = 4000
  grid = (E // eb,)

  def body(x_ref, w_ref, a_ref, e_ref, s0_ref):
    e = jnp.dot(x_ref[...], w_ref[...], preferred_element_type=jnp.float32)
    for t in range(H):
      e_ref[t, ...] = e[:, t * E_OUT:(t + 1) * E_OUT]
    s0_ref[...] = jnp.dot(e, a_ref[...], preferred_element_type=jnp.float32)

  return pl.pallas_call(
      body,
      grid=grid,
      in_specs=[
          pl.BlockSpec((eb, E_IN), lambda i: (i, 0)),
          pl.BlockSpec((E_IN, H * E_OUT), lambda i: (0, 0)),
          pl.BlockSpec((H * E_OUT, H), lambda i: (0, 0)),
      ],
      out_specs=[pl.BlockSpec((H, eb, E_OUT), lambda i: (0, i, 0)),
                 pl.BlockSpec((eb, H), lambda i: (i, 0))],
      out_shape=[jax.ShapeDtypeStruct((H, E, E_OUT), jnp.float32),
                 jax.ShapeDtypeStruct((E, H), jnp.float32)],
  )(edge_fts, we_all, a_e)


def _tc_finalize(ohq, oeq, stats):
  nb = 1000
  grid = (N // nb,)
  dcat = H * (D_OUT + E_OUT)

  def body(ohq_ref, oeq_ref, st_ref, out_ref):
    st = st_ref[...]
    sum_a = st[:, 0:1]
    sum_a2 = st[:, 1:2]
    ef = jnp.float32(E)
    mean = sum_a / ef
    var = sum_a2 / ef - mean * mean
    v = jnp.exp(jnp.clip(var, -2.0, 2.0))
    v = v / jnp.sum(v)
    pieces = []
    for t in range(H):
      sc = v[t, 0]
      for q in range(QP):
        pieces.append(ohq_ref[t * QP + q, ...] * sc)
      pieces.append(oeq_ref[t, ...] * sc)
    out_ref[...] = jnp.concatenate(pieces, axis=1)

  return pl.pallas_call(
      body,
      grid=grid,
      in_specs=[
          pl.BlockSpec((H * QP, nb, DQ), lambda i: (0, i, 0)),
          pl.BlockSpec((H, nb, E_OUT), lambda i: (0, i, 0)),
          pl.BlockSpec((H, L), lambda i: (0, 0)),
      ],
      out_specs=pl.BlockSpec((nb, dcat), lambda i: (i, 0)),
      out_shape=jax.ShapeDtypeStruct((N, dcat), jnp.float32),
  )(ohq, oeq, stats)


@jax.jit
def kernel(node_fts, edge_fts, edges, W_node, W_edge, attn_a):
  src = edges[:, 0].astype(jnp.int32).reshape(NS, NCHUNK, CH)
  dst = edges[:, 1].astype(jnp.int32).reshape(NS, NCHUNK, CH)

  wn_all = jnp.transpose(W_node, (1, 0, 2)).reshape(D_IN, H * D_OUT)
  we_all = jnp.transpose(W_edge, (1, 0, 2)).reshape(E_IN, H * E_OUT)

  a1 = attn_a[:, :D_OUT]
  a2 = attn_a[:, D_OUT:2 * D_OUT]
  a3 = attn_a[:, 2 * D_OUT:2 * D_OUT + E_OUT]
  eye = jnp.eye(H, dtype=jnp.float32)
  a_src_m = (a1[:, :, None] * eye[:, None, :]).reshape(H * D_OUT, H)
  a_dst_m = (a2[:, :, None] * eye[:, None, :]).reshape(H * D_OUT, H)
  a_alpha = jnp.concatenate([a_src_m, a_dst_m], axis=1)
  a_e = (a3[:, :, None] * eye[:, None, :]).reshape(H * E_OUT, H)

  *hqs, alpha = _tc_node_proj(node_fts, wn_all, a_alpha)
  hq = jnp.stack(hqs)
  e_tab, s0 = _tc_edge_proj(edge_fts, we_all, a_e)

  alphaT = alpha.T
  pad = NPAD - N
  as_tab = jnp.pad(alphaT[:H], ((0, 0), (0, pad)))
  ad_tab = jnp.pad(alphaT[H:], ((0, 0), (0, pad)))
  s0q = s0.T.reshape(H, NS, NCHUNK, CH)

  ohq, oeq, stats = _make_sc_call()(src, dst, hq, e_tab, s0q, as_tab, ad_tab)
  return _tc_finalize(ohq, oeq, stats)


# speedup vs baseline: 7.7984x; 1.3336x over previous
import jax
import jax.numpy as jnp
from jax import lax
from jax.experimental import pallas as pl
from jax.experimental.pallas import tpu as pltpu
from jax.experimental.pallas import tpu_sc as plsc

N = 10000
E = 320000
D_IN = 128
D_OUT = 128
E_IN = 16
E_OUT = 16
H = 4
ALPHA = 0.2

NC = 2
NS = 16
L = 16

NPAD = 10240
PERT = NPAD // NS
EPT = E // NS
CH = 80
NCHUNK = EPT // CH
JV = CH // L
CW = 512
SUB = CW // NS
QP = 4
DQ = D_OUT // QP


def _iota16():
  return lax.iota(jnp.int32, L)


def _shift_left(v, k):
  idx = jnp.minimum(_iota16() + k, L - 1)
  return jnp.take_along_axis(v, idx, axis=0, mode="promise_in_bounds")


def _shift_right1(v):
  idx = jnp.maximum(_iota16() - 1, 0)
  return jnp.take_along_axis(v, idx, axis=0, mode="promise_in_bounds")


def _run_masks(dsts):
  it = _iota16()
  first = (it == 0) | (dsts != _shift_right1(dsts))
  last = (it == L - 1) | (dsts != _shift_left(dsts, 1))
  return first, last


def _sc_kernel_body(src_hbm, dst_hbm, hq, e_tab, s0q, as_tab, ad_tab,
                    ohq, oeq, stats,
                    srcb_v, dst_t, sx_t,
                    asrc_v, adst_v, acc_v, comb_v,
                    cstage_v, redtmp_v, rowb_v, erow_v, stat_v, statall_v,
                    outh_s, oute_s, comb_s, red_s, stats_s,
                    sem, sem2, sem3):
  sid = lax.axis_index("s")
  cid = lax.axis_index("c")

  zero16 = jnp.zeros((L,), jnp.float32)

  pltpu.sync_copy(dst_hbm.at[sid], dst_t)

  def src_wait(i):
    slot = lax.rem(i, 2)
    pltpu.make_async_copy(src_hbm.at[sid, i], srcb_v.at[slot], sem2).wait()

    @pl.when(i + 1 < NCHUNK)
    def _():
      pltpu.async_copy(src_hbm.at[sid, i + 1],
                       srcb_v.at[lax.rem(i + 1, 2)], sem2)
    return slot

  def src_prime():
    pltpu.async_copy(src_hbm.at[sid, 0], srcb_v.at[0], sem2)

  def zero_rowb(r, _):
    for k in range(DQ // L):
      rowb_v[0, r, pl.ds(k * L, L)] = zero16
    erow_v[r, :] = zero16
    return 0

  def zero_outh(j, _):
    pltpu.sync_copy(rowb_v.at[0], outh_s.at[pl.ds(sid * PERT + j * CH, CH)])
    return 0

  def process_head(t, _):
    hh = cid * 2 + t
    pltpu.sync_copy(s0q.at[hh, sid], sx_t)
    pltpu.sync_copy(as_tab.at[hh], asrc_v)
    pltpu.sync_copy(ad_tab.at[hh], adst_v)

    lax.fori_loop(0, CH, zero_rowb, 0)

    def zero_acc_copy(j, _):
      base = sid * PERT + j * CH
      pltpu.sync_copy(rowb_v.at[0], outh_s.at[pl.ds(base, CH)])
      pltpu.sync_copy(erow_v, oute_s.at[pl.ds(base, CH)])
      return 0
    lax.fori_loop(0, PERT // CH, zero_acc_copy, 0)

    def init_acc(j, val):
      acc_v[pl.ds(j * L, L)] = val
      return val
    lax.fori_loop(0, NPAD // L, init_acc, jnp.full((L,), -3e38, jnp.float32))

    src_prime()

    def phase_a(i, _):
      slot = src_wait(i)
      for j in range(JV):
        off = pl.ds(j * L, L)
        src16 = srcb_v[slot, off]
        dst16 = dst_t[i, off]
        s = (plsc.load_gather(asrc_v, [src16])
             + plsc.load_gather(adst_v, [dst16])
             + sx_t[i, off])
        s = jnp.where(s > 0, s, ALPHA * s)
        sx_t[i, off] = s
        dsts, ss = plsc.sort_key_val(dst16, s)
        m = ss
        for k in (1, 2, 4, 8):
          idx = jnp.minimum(_iota16() + k, L - 1)
          cand = jnp.take_along_axis(m, idx, axis=0, mode="promise_in_bounds")
          same = dsts == jnp.take_along_axis(dsts, idx, axis=0,
                                             mode="promise_in_bounds")
          m = jnp.where(same, jnp.maximum(m, cand), m)
        first, _ = _run_masks(dsts)
        cur = plsc.load_gather(acc_v, [dsts])
        plsc.store_scatter(acc_v, [dsts], jnp.maximum(cur, m), mask=first)
      return 0
    lax.fori_loop(0, NCHUNK, phase_a, 0)

    def repair_sweep(i, ch):
      for j in range(JV):
        off = pl.ds(j * L, L)
        dst16 = dst_t[i, off]
        s = sx_t[i, off]
        cur = plsc.load_gather(acc_v, [dst16])
        need = s > cur
        plsc.store_scatter(acc_v, [dst16], jnp.maximum(cur, s), mask=need)
        ch = ch | jnp.any(need)
      return ch

    lax.while_loop(
        lambda c: c,
        lambda c: lax.fori_loop(0, NCHUNK, repair_sweep, jnp.bool_(False)),
        jnp.bool_(True))

    def combine(op_is_max):
      plsc.subcore_barrier()

      def round_body(r, _):
        pltpu.sync_copy(acc_v.at[pl.ds(r * CW, CW)],
                        comb_s.at[pl.ds(sid * CW, CW)])
        plsc.subcore_barrier()
        for t in range(NS):
          pltpu.sync_copy(comb_s.at[pl.ds(t * CW + sid * SUB, SUB)],
                          cstage_v.at[pl.ds(t * SUB, SUB)])
        for j in range(SUB // L):
          v = cstage_v[pl.ds(j * L, L)]
          for t in range(1, NS):
            w = cstage_v[pl.ds(t * SUB + j * L, L)]
            v = jnp.maximum(v, w) if op_is_max else v + w
          redtmp_v[pl.ds(j * L, L)] = v
        pltpu.sync_copy(redtmp_v, red_s.at[pl.ds(r * CW + sid * SUB, SUB)])
        plsc.subcore_barrier()
        return 0
      lax.fori_loop(0, NPAD // CW, round_body, 0)
      pltpu.sync_copy(red_s, comb_v)

    combine(op_is_max=True)

    lax.fori_loop(0, NPAD // L, init_acc, jnp.zeros((L,), jnp.float32))

    def phase_b(i, _):
      for j in range(JV):
        off = pl.ds(j * L, L)
        dst16 = dst_t[i, off]
        m16 = plsc.load_gather(comb_v, [dst16])
        ex = jnp.exp(jnp.minimum(sx_t[i, off] - m16, 80.0))
        sx_t[i, off] = ex
        dsts, exs = plsc.sort_key_val(dst16, ex)
        c = plsc.cumsum(exs)
        prev = jnp.where(_iota16() == 0, 0.0, _shift_right1(c))
        first, last = _run_masks(dsts)
        base = plsc.cummax(jnp.where(first, prev, 0.0))
        run_sum = c - base
        plsc.addupdate_scatter(acc_v, [dsts], run_sum, mask=last)
      return 0
    lax.fori_loop(0, NCHUNK, phase_b, 0)

    combine(op_is_max=False)

    def c_pipeline(plane, with_stats):
      ssem = (sem2, sem3)

      pltpu.async_copy(src_hbm.at[sid, 0], srcb_v.at[0], sem2)
      pltpu.async_copy(src_hbm.at[sid, 1], srcb_v.at[1], sem3)
      pltpu.make_async_copy(src_hbm.at[sid, 0], srcb_v.at[0], sem2).wait()
      pltpu.async_copy(hq.at[plane].at[srcb_v.at[0]], rowb_v.at[0], sem)

      def body2(i2, carry):
        sa, sa2 = carry
        for par in (0, 1):
          oth = 1 - par
          i = 2 * i2 + par
          if with_stats:
            for j in range(JV):
              off = pl.ds(j * L, L)
              dst16 = dst_t[i, off]
              den16 = plsc.load_gather(comb_v, [dst16])
              a = sx_t[i, off] / (den16 + 1e-9)
              sx_t[i, off] = a
              sa = sa + a
              sa2 = sa2 + a * a
            eoff = sid * EPT + i * CH
            pltpu.sync_copy(e_tab.at[hh, pl.ds(eoff, CH)], erow_v)

          pltpu.make_async_copy(hq.at[plane].at[srcb_v.at[par]],
                                rowb_v.at[par], sem).wait()

          @pl.when(i + 2 < NCHUNK)
          def _():
            pltpu.async_copy(src_hbm.at[sid, i + 2], srcb_v.at[par],
                             ssem[par])

          @pl.when(i + 1 < NCHUNK)
          def _():
            pltpu.make_async_copy(src_hbm.at[sid, i + 1], srcb_v.at[oth],
                                  ssem[oth]).wait()
            pltpu.async_copy(hq.at[plane].at[srcb_v.at[oth]],
                             rowb_v.at[oth], sem)

          ri = lax.broadcast(i, (L,))

          def scale_row(r, _):
            ar = plsc.load_gather(sx_t, [ri, lax.broadcast(r, (L,))])
            for k in range(DQ // L):
              o = pl.ds(k * L, L)
              rowb_v[par, r, o] = rowb_v[par, r, o] * ar
            if with_stats:
              erow_v[r, :] = erow_v[r, :] * ar
            return 0
          lax.fori_loop(0, CH, scale_row, 0)

          pltpu.sync_copy(rowb_v.at[par], outh_s.at[dst_t.at[i]], add=True)
          if with_stats:
            pltpu.sync_copy(erow_v, oute_s.at[dst_t.at[i]], add=True)
        return (sa, sa2)

      return lax.fori_loop(0, NCHUNK // 2, body2,
                           (jnp.zeros((L,), jnp.float32),
                            jnp.zeros((L,), jnp.float32)))

    sa, sa2 = c_pipeline(hh * QP, with_stats=True)

    it = _iota16()
    stat_v[:] = jnp.where(it == 0, jnp.sum(sa),
                          jnp.where(it == 1, jnp.sum(sa2), 0.0))
    pltpu.sync_copy(stat_v, stats_s.at[pl.ds(sid * L, L)])
    plsc.subcore_barrier()

    @pl.when(sid == 0)
    def _():
      pltpu.sync_copy(stats_s, statall_v)
      r = statall_v[pl.ds(0, L)]
      for t in range(1, NS):
        r = r + statall_v[pl.ds(t * L, L)]
      stat_v[:] = r
      pltpu.sync_copy(stat_v, stats.at[hh])

    base = sid * PERT

    pltpu.sync_copy(oute_s.at[pl.ds(base, PERT)],
                    oeq.at[hh, pl.ds(base, PERT)])

    def h_pass(q, _):
      plsc.subcore_barrier()
      pltpu.sync_copy(outh_s.at[pl.ds(base, PERT)],
                      ohq.at[hh * QP + q - 1, pl.ds(base, PERT)])
      lax.fori_loop(0, CH, zero_rowb, 0)
      lax.fori_loop(0, PERT // CH, zero_outh, 0)
      plsc.subcore_barrier()
      c_pipeline(hh * QP + q, with_stats=False)
      return 0
    lax.fori_loop(1, QP, h_pass, 0)

    plsc.subcore_barrier()
    pltpu.sync_copy(outh_s.at[pl.ds(base, PERT)],
                    ohq.at[hh * QP + QP - 1, pl.ds(base, PERT)])
    plsc.subcore_barrier()
    return 0

  lax.fori_loop(0, H // NC, process_head, 0)


def _make_sc_call():
  mesh = plsc.VectorSubcoreMesh(core_axis_name="c", subcore_axis_name="s",
                                num_cores=NC, num_subcores=NS)
  out_type = (
      jax.ShapeDtypeStruct((H * QP, NPAD, DQ), jnp.float32),
      jax.ShapeDtypeStruct((H, NPAD, E_OUT), jnp.float32),
      jax.ShapeDtypeStruct((H, L), jnp.float32),
  )
  scratch = [
      pltpu.VMEM((2, CH), jnp.int32),
      pltpu.VMEM((NCHUNK, CH), jnp.int32),
      pltpu.VMEM((NCHUNK, CH), jnp.float32),
      pltpu.VMEM((NPAD,), jnp.float32),
      pltpu.VMEM((NPAD,), jnp.float32),
      pltpu.VMEM((NPAD,), jnp.float32),
      pltpu.VMEM((NPAD,), jnp.float32),
      pltpu.VMEM((NS * SUB,), jnp.float32),
      pltpu.VMEM((SUB,), jnp.float32),
      pltpu.VMEM((2, CH, DQ), jnp.float32),
      pltpu.VMEM((CH, E_OUT), jnp.float32),
      pltpu.VMEM((L,), jnp.float32),
      pltpu.VMEM((NS * L,), jnp.float32),
      pltpu.VMEM_SHARED((NPAD, DQ), jnp.float32),
      pltpu.VMEM_SHARED((NPAD, E_OUT), jnp.float32),
      pltpu.VMEM_SHARED((NS * CW,), jnp.float32),
      pltpu.VMEM_SHARED((NPAD,), jnp.float32),
      pltpu.VMEM_SHARED((NS * L,), jnp.float32),
      pltpu.SemaphoreType.DMA,
      pltpu.SemaphoreType.DMA,
      pltpu.SemaphoreType.DMA,
  ]
  return pl.kernel(_sc_kernel_body, out_type=out_type, mesh=mesh,
                   scratch_types=scratch,
                   compiler_params=pltpu.CompilerParams(
                       needs_layout_passes=False,
                       use_tc_tiling_on_sc=False))


def _tc_node_proj(node_fts, wn_all, a_alpha):
  nb = 1000
  grid = (N // nb,)

  def body(x_ref, w_ref, a_ref, *out_refs):
    h = jnp.dot(x_ref[...], w_ref[...], preferred_element_type=jnp.float32)
    for t in range(H):
      for q in range(QP):
        c0 = t * D_OUT + q * DQ
        out_refs[t * QP + q][...] = h[:, c0:c0 + DQ]
    out_refs[H * QP][...] = jnp.dot(h, a_ref[...],
                                    preferred_element_type=jnp.float32)

  return pl.pallas_call(
      body,
      grid=grid,
      in_specs=[
          pl.BlockSpec((nb, D_IN), lambda i: (i, 0)),
          pl.BlockSpec((D_IN, H * D_OUT), lambda i: (0, 0)),
          pl.BlockSpec((H * D_OUT, 2 * H), lambda i: (0, 0)),
      ],
      out_specs=[pl.BlockSpec((nb, DQ), lambda i: (i, 0))
                 for _ in range(H * QP)]
      + [pl.BlockSpec((nb, 2 * H), lambda i: (i, 0))],
      out_shape=[jax.ShapeDtypeStruct((N, DQ), jnp.float32)
                 for _ in range(H * QP)]
      + [jax.ShapeDtypeStruct((N, 2 * H), jnp.float32)],
  )(node_fts, wn_all, a_alpha)


def _tc_edge_proj(edge_fts, we_all, a_e):
  eb = 4000
  grid = (E // eb,)

  def body(x_ref, w_ref, a_ref, e_ref, s0_ref):
    e = jnp.dot(x_ref[...], w_ref[...], preferred_element_type=jnp.float32)
    for t in range(H):
      e_ref[t, ...] = e[:, t * E_OUT:(t + 1) * E_OUT]
    s0_ref[...] = jnp.dot(e, a_ref[...], preferred_element_type=jnp.float32)

  return pl.pallas_call(
      body,
      grid=grid,
      in_specs=[
          pl.BlockSpec((eb, E_IN), lambda i: (i, 0)),
          pl.BlockSpec((E_IN, H * E_OUT), lambda i: (0, 0)),
          pl.BlockSpec((H * E_OUT, H), lambda i: (0, 0)),
      ],
      out_specs=[pl.BlockSpec((H, eb, E_OUT), lambda i: (0, i, 0)),
                 pl.BlockSpec((eb, H), lambda i: (i, 0))],
      out_shape=[jax.ShapeDtypeStruct((H, E, E_OUT), jnp.float32),
                 jax.ShapeDtypeStruct((E, H), jnp.float32)],
  )(edge_fts, we_all, a_e)


def _tc_finalize(ohq, oeq, stats):
  nb = 1000
  grid = (N // nb,)
  dcat = H * (D_OUT + E_OUT)

  def body(ohq_ref, oeq_ref, st_ref, out_ref):
    st = st_ref[...]
    sum_a = st[:, 0:1]
    sum_a2 = st[:, 1:2]
    ef = jnp.float32(E)
    mean = sum_a / ef
    var = sum_a2 / ef - mean * mean
    v = jnp.exp(jnp.clip(var, -2.0, 2.0))
    v = v / jnp.sum(v)
    pieces = []
    for t in range(H):
      sc = v[t, 0]
      for q in range(QP):
        pieces.append(ohq_ref[t * QP + q, ...] * sc)
      pieces.append(oeq_ref[t, ...] * sc)
    out_ref[...] = jnp.concatenate(pieces, axis=1)

  return pl.pallas_call(
      body,
      grid=grid,
      in_specs=[
          pl.BlockSpec((H * QP, nb, DQ), lambda i: (0, i, 0)),
          pl.BlockSpec((H, nb, E_OUT), lambda i: (0, i, 0)),
          pl.BlockSpec((H, L), lambda i: (0, 0)),
      ],
      out_specs=pl.BlockSpec((nb, dcat), lambda i: (i, 0)),
      out_shape=jax.ShapeDtypeStruct((N, dcat), jnp.float32),
  )(ohq, oeq, stats)


@jax.jit
def kernel(node_fts, edge_fts, edges, W_node, W_edge, attn_a):
  src = edges[:, 0].astype(jnp.int32).reshape(NS, NCHUNK, CH)
  dst = edges[:, 1].astype(jnp.int32).reshape(NS, NCHUNK, CH)

  wn_all = jnp.transpose(W_node, (1, 0, 2)).reshape(D_IN, H * D_OUT)
  we_all = jnp.transpose(W_edge, (1, 0, 2)).reshape(E_IN, H * E_OUT)

  a1 = attn_a[:, :D_OUT]
  a2 = attn_a[:, D_OUT:2 * D_OUT]
  a3 = attn_a[:, 2 * D_OUT:2 * D_OUT + E_OUT]
  eye = jnp.eye(H, dtype=jnp.float32)
  a_src_m = (a1[:, :, None] * eye[:, None, :]).reshape(H * D_OUT, H)
  a_dst_m = (a2[:, :, None] * eye[:, None, :]).reshape(H * D_OUT, H)
  a_alpha = jnp.concatenate([a_src_m, a_dst_m], axis=1)
  a_e = (a3[:, :, None] * eye[:, None, :]).reshape(H * E_OUT, H)

  *hqs, alpha = _tc_node_proj(node_fts, wn_all, a_alpha)
  hq = jnp.stack(hqs)
  e_tab, s0 = _tc_edge_proj(edge_fts, we_all, a_e)

  alphaT = alpha.T
  pad = NPAD - N
  as_tab = jnp.pad(alphaT[:H], ((0, 0), (0, pad)))
  ad_tab = jnp.pad(alphaT[H:], ((0, 0), (0, pad)))
  s0q = s0.T.reshape(H, NS, NCHUNK, CH)

  ohq, oeq, stats = _make_sc_call()(src, dst, hq, e_tab, s0q, as_tab, ad_tab)
  return _tc_finalize(ohq, oeq, stats)


# speedup vs baseline: 7.8934x; 1.0122x over previous
import jax
import jax.numpy as jnp
from jax import lax
from jax.experimental import pallas as pl
from jax.experimental.pallas import tpu as pltpu
from jax.experimental.pallas import tpu_sc as plsc

N = 10000
E = 320000
D_IN = 128
D_OUT = 128
E_IN = 16
E_OUT = 16
H = 4
ALPHA = 0.2

NC = 2
NS = 16
L = 16

NPAD = 10240
PERT = NPAD // NS
EPT = E // NS
CH = 80
NCHUNK = EPT // CH
JV = CH // L
CW = 512
SUB = CW // NS
QP = 4
DQ = D_OUT // QP


def _iota16():
  return lax.iota(jnp.int32, L)


def _shift_left(v, k):
  idx = jnp.minimum(_iota16() + k, L - 1)
  return jnp.take_along_axis(v, idx, axis=0, mode="promise_in_bounds")


def _shift_right1(v):
  idx = jnp.maximum(_iota16() - 1, 0)
  return jnp.take_along_axis(v, idx, axis=0, mode="promise_in_bounds")


def _run_masks(dsts):
  it = _iota16()
  first = (it == 0) | (dsts != _shift_right1(dsts))
  last = (it == L - 1) | (dsts != _shift_left(dsts, 1))
  return first, last


def _sc_kernel_body(src_hbm, dst_hbm, hq, e_tab, s0q, as_tab, ad_tab,
                    ohq, oeq, stats,
                    srcb_v, dst_t, sx_t,
                    asrc_v, adst_v, acc_v, comb_v,
                    cstage_v, redtmp_v, rowb_v, erow_v, stat_v, statall_v,
                    outh_s, oute_s, comb_s, red_s, stats_s,
                    sem, sem2, sem3, sem4):
  sid = lax.axis_index("s")
  cid = lax.axis_index("c")

  zero16 = jnp.zeros((L,), jnp.float32)

  pltpu.sync_copy(dst_hbm.at[sid], dst_t)

  def src_wait(i):
    slot = lax.rem(i, 2)
    pltpu.make_async_copy(src_hbm.at[sid, i], srcb_v.at[slot], sem2).wait()

    @pl.when(i + 1 < NCHUNK)
    def _():
      pltpu.async_copy(src_hbm.at[sid, i + 1],
                       srcb_v.at[lax.rem(i + 1, 2)], sem2)
    return slot

  def src_prime():
    pltpu.async_copy(src_hbm.at[sid, 0], srcb_v.at[0], sem2)

  def zero_rowb(r, _):
    for k in range(DQ // L):
      rowb_v[0, r, pl.ds(k * L, L)] = zero16
    erow_v[r, :] = zero16
    return 0

  def zero_outh(j, _):
    pltpu.sync_copy(rowb_v.at[0], outh_s.at[pl.ds(sid * PERT + j * CH, CH)])
    return 0

  def process_head(t, _):
    hh = cid * 2 + t
    pltpu.sync_copy(s0q.at[hh, sid], sx_t)
    pltpu.sync_copy(as_tab.at[hh], asrc_v)
    pltpu.sync_copy(ad_tab.at[hh], adst_v)

    lax.fori_loop(0, CH, zero_rowb, 0)

    def zero_acc_copy(j, _):
      base = sid * PERT + j * CH
      pltpu.sync_copy(rowb_v.at[0], outh_s.at[pl.ds(base, CH)])
      pltpu.sync_copy(erow_v, oute_s.at[pl.ds(base, CH)])
      return 0
    lax.fori_loop(0, PERT // CH, zero_acc_copy, 0)

    def init_acc(j, val):
      acc_v[pl.ds(j * L, L)] = val
      return val
    lax.fori_loop(0, NPAD // L, init_acc, jnp.full((L,), -3e38, jnp.float32))

    src_prime()

    def phase_a(i, _):
      slot = src_wait(i)
      for j in range(JV):
        off = pl.ds(j * L, L)
        src16 = srcb_v[slot, off]
        dst16 = dst_t[i, off]
        s = (plsc.load_gather(asrc_v, [src16])
             + plsc.load_gather(adst_v, [dst16])
             + sx_t[i, off])
        s = jnp.where(s > 0, s, ALPHA * s)
        sx_t[i, off] = s
        dsts, ss = plsc.sort_key_val(dst16, s)
        m = ss
        for k in (1, 2, 4, 8):
          idx = jnp.minimum(_iota16() + k, L - 1)
          cand = jnp.take_along_axis(m, idx, axis=0, mode="promise_in_bounds")
          same = dsts == jnp.take_along_axis(dsts, idx, axis=0,
                                             mode="promise_in_bounds")
          m = jnp.where(same, jnp.maximum(m, cand), m)
        first, _ = _run_masks(dsts)
        cur = plsc.load_gather(acc_v, [dsts])
        plsc.store_scatter(acc_v, [dsts], jnp.maximum(cur, m), mask=first)
      return 0
    lax.fori_loop(0, NCHUNK, phase_a, 0)

    def repair_sweep(i, ch):
      for j in range(JV):
        off = pl.ds(j * L, L)
        dst16 = dst_t[i, off]
        s = sx_t[i, off]
        cur = plsc.load_gather(acc_v, [dst16])
        need = s > cur
        plsc.store_scatter(acc_v, [dst16], jnp.maximum(cur, s), mask=need)
        ch = ch | jnp.any(need)
      return ch

    lax.while_loop(
        lambda c: c,
        lambda c: lax.fori_loop(0, NCHUNK, repair_sweep, jnp.bool_(False)),
        jnp.bool_(True))

    def combine(op_is_max):
      plsc.subcore_barrier()

      def round_body(r, _):
        pltpu.sync_copy(acc_v.at[pl.ds(r * CW, CW)],
                        comb_s.at[pl.ds(sid * CW, CW)])
        plsc.subcore_barrier()
        for t in range(NS):
          pltpu.sync_copy(comb_s.at[pl.ds(t * CW + sid * SUB, SUB)],
                          cstage_v.at[pl.ds(t * SUB, SUB)])
        for j in range(SUB // L):
          v = cstage_v[pl.ds(j * L, L)]
          for t in range(1, NS):
            w = cstage_v[pl.ds(t * SUB + j * L, L)]
            v = jnp.maximum(v, w) if op_is_max else v + w
          redtmp_v[pl.ds(j * L, L)] = v
        pltpu.sync_copy(redtmp_v, red_s.at[pl.ds(r * CW + sid * SUB, SUB)])
        plsc.subcore_barrier()
        return 0
      lax.fori_loop(0, NPAD // CW, round_body, 0)
      pltpu.sync_copy(red_s, comb_v)

    combine(op_is_max=True)

    lax.fori_loop(0, NPAD // L, init_acc, jnp.zeros((L,), jnp.float32))

    def phase_b(i, _):
      for j in range(JV):
        off = pl.ds(j * L, L)
        dst16 = dst_t[i, off]
        m16 = plsc.load_gather(comb_v, [dst16])
        ex = jnp.exp(jnp.minimum(sx_t[i, off] - m16, 80.0))
        sx_t[i, off] = ex
        dsts, exs = plsc.sort_key_val(dst16, ex)
        c = plsc.cumsum(exs)
        prev = jnp.where(_iota16() == 0, 0.0, _shift_right1(c))
        first, last = _run_masks(dsts)
        base = plsc.cummax(jnp.where(first, prev, 0.0))
        run_sum = c - base
        plsc.addupdate_scatter(acc_v, [dsts], run_sum, mask=last)
      return 0
    lax.fori_loop(0, NCHUNK, phase_b, 0)

    combine(op_is_max=False)

    def c_pipeline(plane, with_stats):
      ssem = (sem2, sem3)

      pltpu.async_copy(src_hbm.at[sid, 0], srcb_v.at[0], sem2)
      pltpu.async_copy(src_hbm.at[sid, 1], srcb_v.at[1], sem3)
      pltpu.make_async_copy(src_hbm.at[sid, 0], srcb_v.at[0], sem2).wait()
      pltpu.async_copy(hq.at[plane].at[srcb_v.at[0]], rowb_v.at[0], sem)

      def body2(i2, carry):
        sa, sa2 = carry
        for par in (0, 1):
          oth = 1 - par
          i = 2 * i2 + par
          if with_stats:
            for j in range(JV):
              off = pl.ds(j * L, L)
              dst16 = dst_t[i, off]
              den16 = plsc.load_gather(comb_v, [dst16])
              a = sx_t[i, off] / (den16 + 1e-9)
              sx_t[i, off] = a
              sa = sa + a
              sa2 = sa2 + a * a
            eoff = sid * EPT + i * CH
            pltpu.sync_copy(e_tab.at[hh, pl.ds(eoff, CH)], erow_v)

          pltpu.make_async_copy(hq.at[plane].at[srcb_v.at[par]],
                                rowb_v.at[par], sem).wait()

          @pl.when(i + 2 < NCHUNK)
          def _():
            pltpu.async_copy(src_hbm.at[sid, i + 2], srcb_v.at[par],
                             ssem[par])

          @pl.when(i >= 1)
          def _():
            pltpu.make_async_copy(rowb_v.at[oth],
                                  outh_s.at[dst_t.at[i - 1]], sem4).wait()

          @pl.when(i + 1 < NCHUNK)
          def _():
            pltpu.make_async_copy(src_hbm.at[sid, i + 1], srcb_v.at[oth],
                                  ssem[oth]).wait()
            pltpu.async_copy(hq.at[plane].at[srcb_v.at[oth]],
                             rowb_v.at[oth], sem)

          ri = lax.broadcast(i, (L,))

          def scale_row(r, _):
            ar = plsc.load_gather(sx_t, [ri, lax.broadcast(r, (L,))])
            for k in range(DQ // L):
              o = pl.ds(k * L, L)
              rowb_v[par, r, o] = rowb_v[par, r, o] * ar
            if with_stats:
              erow_v[r, :] = erow_v[r, :] * ar
            return 0
          lax.fori_loop(0, CH, scale_row, 0)

          pltpu.async_copy(rowb_v.at[par], outh_s.at[dst_t.at[i]], sem4,
                           add=True)
          if with_stats:
            pltpu.sync_copy(erow_v, oute_s.at[dst_t.at[i]], add=True)
        return (sa, sa2)

      out = lax.fori_loop(0, NCHUNK // 2, body2,
                          (jnp.zeros((L,), jnp.float32),
                           jnp.zeros((L,), jnp.float32)))
      pltpu.make_async_copy(rowb_v.at[1],
                            outh_s.at[dst_t.at[NCHUNK - 1]], sem4).wait()
      return out

    sa, sa2 = c_pipeline(hh * QP, with_stats=True)

    it = _iota16()
    stat_v[:] = jnp.where(it == 0, jnp.sum(sa),
                          jnp.where(it == 1, jnp.sum(sa2), 0.0))
    pltpu.sync_copy(stat_v, stats_s.at[pl.ds(sid * L, L)])
    plsc.subcore_barrier()

    @pl.when(sid == 0)
    def _():
      pltpu.sync_copy(stats_s, statall_v)
      r = statall_v[pl.ds(0, L)]
      for t in range(1, NS):
        r = r + statall_v[pl.ds(t * L, L)]
      stat_v[:] = r
      pltpu.sync_copy(stat_v, stats.at[hh])

    base = sid * PERT

    pltpu.sync_copy(oute_s.at[pl.ds(base, PERT)],
                    oeq.at[hh, pl.ds(base, PERT)])

    def h_pass(q, _):
      plsc.subcore_barrier()
      pltpu.sync_copy(outh_s.at[pl.ds(base, PERT)],
                      ohq.at[hh * QP + q - 1, pl.ds(base, PERT)])
      lax.fori_loop(0, CH, zero_rowb, 0)
      lax.fori_loop(0, PERT // CH, zero_outh, 0)
      plsc.subcore_barrier()
      c_pipeline(hh * QP + q, with_stats=False)
      return 0
    lax.fori_loop(1, QP, h_pass, 0)

    plsc.subcore_barrier()
    pltpu.sync_copy(outh_s.at[pl.ds(base, PERT)],
                    ohq.at[hh * QP + QP - 1, pl.ds(base, PERT)])
    plsc.subcore_barrier()
    return 0

  lax.fori_loop(0, H // NC, process_head, 0)


def _make_sc_call():
  mesh = plsc.VectorSubcoreMesh(core_axis_name="c", subcore_axis_name="s",
                                num_cores=NC, num_subcores=NS)
  out_type = (
      jax.ShapeDtypeStruct((H * QP, NPAD, DQ), jnp.float32),
      jax.ShapeDtypeStruct((H, NPAD, E_OUT), jnp.float32),
      jax.ShapeDtypeStruct((H, L), jnp.float32),
  )
  scratch = [
      pltpu.VMEM((2, CH), jnp.int32),
      pltpu.VMEM((NCHUNK, CH), jnp.int32),
      pltpu.VMEM((NCHUNK, CH), jnp.float32),
      pltpu.VMEM((NPAD,), jnp.float32),
      pltpu.VMEM((NPAD,), jnp.float32),
      pltpu.VMEM((NPAD,), jnp.float32),
      pltpu.VMEM((NPAD,), jnp.float32),
      pltpu.VMEM((NS * SUB,), jnp.float32),
      pltpu.VMEM((SUB,), jnp.float32),
      pltpu.VMEM((2, CH, DQ), jnp.float32),
      pltpu.VMEM((CH, E_OUT), jnp.float32),
      pltpu.VMEM((L,), jnp.float32),
      pltpu.VMEM((NS * L,), jnp.float32),
      pltpu.VMEM_SHARED((NPAD, DQ), jnp.float32),
      pltpu.VMEM_SHARED((NPAD, E_OUT), jnp.float32),
      pltpu.VMEM_SHARED((NS * CW,), jnp.float32),
      pltpu.VMEM_SHARED((NPAD,), jnp.float32),
      pltpu.VMEM_SHARED((NS * L,), jnp.float32),
      pltpu.SemaphoreType.DMA,
      pltpu.SemaphoreType.DMA,
      pltpu.SemaphoreType.DMA,
      pltpu.SemaphoreType.DMA,
  ]
  return pl.kernel(_sc_kernel_body, out_type=out_type, mesh=mesh,
                   scratch_types=scratch,
                   compiler_params=pltpu.CompilerParams(
                       needs_layout_passes=False,
                       use_tc_tiling_on_sc=False))


def _tc_node_proj(node_fts, wn_all, a_alpha):
  nb = 1000
  grid = (N // nb,)

  def body(x_ref, w_ref, a_ref, *out_refs):
    h = jnp.dot(x_ref[...], w_ref[...], preferred_element_type=jnp.float32)
    for t in range(H):
      for q in range(QP):
        c0 = t * D_OUT + q * DQ
        out_refs[t * QP + q][...] = h[:, c0:c0 + DQ]
    out_refs[H * QP][...] = jnp.dot(h, a_ref[...],
                                    preferred_element_type=jnp.float32)

  return pl.pallas_call(
      body,
      grid=grid,
      in_specs=[
          pl.BlockSpec((nb, D_IN), lambda i: (i, 0)),
          pl.BlockSpec((D_IN, H * D_OUT), lambda i: (0, 0)),
          pl.BlockSpec((H * D_OUT, 2 * H), lambda i: (0, 0)),
      ],
      out_specs=[pl.BlockSpec((nb, DQ), lambda i: (i, 0))
                 for _ in range(H * QP)]
      + [pl.BlockSpec((nb, 2 * H), lambda i: (i, 0))],
      out_shape=[jax.ShapeDtypeStruct((N, DQ), jnp.float32)
                 for _ in range(H * QP)]
      + [jax.ShapeDtypeStruct((N, 2 * H), jnp.float32)],
  )(node_fts, wn_all, a_alpha)


def _tc_edge_proj(edge_fts, we_all, a_e):
  eb = 4000
  grid = (E // eb,)

  def body(x_ref, w_ref, a_ref, e_ref, s0_ref):
    e = jnp.dot(x_ref[...], w_ref[...], preferred_element_type=jnp.float32)
    for t in range(H):
      e_ref[t, ...] = e[:, t * E_OUT:(t + 1) * E_OUT]
    s0_ref[...] = jnp.dot(e, a_ref[...], preferred_element_type=jnp.float32)

  return pl.pallas_call(
      body,
      grid=grid,
      in_specs=[
          pl.BlockSpec((eb, E_IN), lambda i: (i, 0)),
          pl.BlockSpec((E_IN, H * E_OUT), lambda i: (0, 0)),
          pl.BlockSpec((H * E_OUT, H), lambda i: (0, 0)),
      ],
      out_specs=[pl.BlockSpec((H, eb, E_OUT), lambda i: (0, i, 0)),
                 pl.BlockSpec((eb, H), lambda i: (i, 0))],
      out_shape=[jax.ShapeDtypeStruct((H, E, E_OUT), jnp.float32),
                 jax.ShapeDtypeStruct((E, H), jnp.float32)],
  )(edge_fts, we_all, a_e)


def _tc_finalize(ohq, oeq, stats):
  nb = 1000
  grid = (N // nb,)
  dcat = H * (D_OUT + E_OUT)

  def body(ohq_ref, oeq_ref, st_ref, out_ref):
    st = st_ref[...]
    sum_a = st[:, 0:1]
    sum_a2 = st[:, 1:2]
    ef = jnp.float32(E)
    mean = sum_a / ef
    var = sum_a2 / ef - mean * mean
    v = jnp.exp(jnp.clip(var, -2.0, 2.0))
    v = v / jnp.sum(v)
    pieces = []
    for t in range(H):
      sc = v[t, 0]
      for q in range(QP):
        pieces.append(ohq_ref[t * QP + q, ...] * sc)
      pieces.append(oeq_ref[t, ...] * sc)
    out_ref[...] = jnp.concatenate(pieces, axis=1)

  return pl.pallas_call(
      body,
      grid=grid,
      in_specs=[
          pl.BlockSpec((H * QP, nb, DQ), lambda i: (0, i, 0)),
          pl.BlockSpec((H, nb, E_OUT), lambda i: (0, i, 0)),
          pl.BlockSpec((H, L), lambda i: (0, 0)),
      ],
      out_specs=pl.BlockSpec((nb, dcat), lambda i: (i, 0)),
      out_shape=jax.ShapeDtypeStruct((N, dcat), jnp.float32),
  )(ohq, oeq, stats)


@jax.jit
def kernel(node_fts, edge_fts, edges, W_node, W_edge, attn_a):
  src = edges[:, 0].astype(jnp.int32).reshape(NS, NCHUNK, CH)
  dst = edges[:, 1].astype(jnp.int32).reshape(NS, NCHUNK, CH)

  wn_all = jnp.transpose(W_node, (1, 0, 2)).reshape(D_IN, H * D_OUT)
  we_all = jnp.transpose(W_edge, (1, 0, 2)).reshape(E_IN, H * E_OUT)

  a1 = attn_a[:, :D_OUT]
  a2 = attn_a[:, D_OUT:2 * D_OUT]
  a3 = attn_a[:, 2 * D_OUT:2 * D_OUT + E_OUT]
  eye = jnp.eye(H, dtype=jnp.float32)
  a_src_m = (a1[:, :, None] * eye[:, None, :]).reshape(H * D_OUT, H)
  a_dst_m = (a2[:, :, None] * eye[:, None, :]).reshape(H * D_OUT, H)
  a_alpha = jnp.concatenate([a_src_m, a_dst_m], axis=1)
  a_e = (a3[:, :, None] * eye[:, None, :]).reshape(H * E_OUT, H)

  *hqs, alpha = _tc_node_proj(node_fts, wn_all, a_alpha)
  hq = jnp.stack(hqs)
  e_tab, s0 = _tc_edge_proj(edge_fts, we_all, a_e)

  alphaT = alpha.T
  pad = NPAD - N
  as_tab = jnp.pad(alphaT[:H], ((0, 0), (0, pad)))
  ad_tab = jnp.pad(alphaT[H:], ((0, 0), (0, pad)))
  s0q = s0.T.reshape(H, NS, NCHUNK, CH)

  ohq, oeq, stats = _make_sc_call()(src, dst, hq, e_tab, s0q, as_tab, ad_tab)
  return _tc_finalize(ohq, oeq, stats)


# speedup vs baseline: 8.0291x; 1.0172x over previous
import jax
import jax.numpy as jnp
from jax import lax
from jax.experimental import pallas as pl
from jax.experimental.pallas import tpu as pltpu
from jax.experimental.pallas import tpu_sc as plsc

N = 10000
E = 320000
D_IN = 128
D_OUT = 128
E_IN = 16
E_OUT = 16
H = 4
ALPHA = 0.2

NC = 2
NS = 16
L = 16

NPAD = 10240
PERT = NPAD // NS
EPT = E // NS
CH = 80
NCHUNK = EPT // CH
JV = CH // L
CW = 512
SUB = CW // NS
QP = 4
DQ = D_OUT // QP


def _iota16():
  return lax.iota(jnp.int32, L)


def _shift_left(v, k):
  idx = jnp.minimum(_iota16() + k, L - 1)
  return jnp.take_along_axis(v, idx, axis=0, mode="promise_in_bounds")


def _shift_right1(v):
  idx = jnp.maximum(_iota16() - 1, 0)
  return jnp.take_along_axis(v, idx, axis=0, mode="promise_in_bounds")


def _run_masks(dsts):
  it = _iota16()
  first = (it == 0) | (dsts != _shift_right1(dsts))
  last = (it == L - 1) | (dsts != _shift_left(dsts, 1))
  return first, last


def _sc_kernel_body(src_hbm, dst_hbm, hq, e_tab, s0q, as_tab, ad_tab,
                    ohq, oeq, stats,
                    srcb_v, dst_t, sx_t,
                    asrc_v, adst_v, acc_v, comb_v,
                    cstage_v, redtmp_v, rowb_v, erow_v, stat_v, statall_v,
                    outh_s, oute_s, comb_s, red_s, stats_s,
                    sem, sem2, sem3, sem4):
  sid = lax.axis_index("s")
  cid = lax.axis_index("c")

  zero16 = jnp.zeros((L,), jnp.float32)

  pltpu.sync_copy(dst_hbm.at[sid], dst_t)

  def src_wait(i):
    slot = lax.rem(i, 2)
    pltpu.make_async_copy(src_hbm.at[sid, i], srcb_v.at[slot], sem2).wait()

    @pl.when(i + 1 < NCHUNK)
    def _():
      pltpu.async_copy(src_hbm.at[sid, i + 1],
                       srcb_v.at[lax.rem(i + 1, 2)], sem2)
    return slot

  def src_prime():
    pltpu.async_copy(src_hbm.at[sid, 0], srcb_v.at[0], sem2)

  def zero_rowb(r, _):
    for k in range(DQ // L):
      rowb_v[0, r, pl.ds(k * L, L)] = zero16
    erow_v[r, :] = zero16
    return 0

  def zero_outh(j, _):
    pltpu.sync_copy(rowb_v.at[0], outh_s.at[pl.ds(sid * PERT + j * CH, CH)])
    return 0

  def process_head(t, _):
    hh = cid * 2 + t
    pltpu.sync_copy(s0q.at[hh, sid], sx_t)
    pltpu.sync_copy(as_tab.at[hh], asrc_v)
    pltpu.sync_copy(ad_tab.at[hh], adst_v)

    lax.fori_loop(0, CH, zero_rowb, 0)

    def zero_acc_copy(j, _):
      base = sid * PERT + j * CH
      pltpu.sync_copy(rowb_v.at[0], outh_s.at[pl.ds(base, CH)])
      pltpu.sync_copy(erow_v, oute_s.at[pl.ds(base, CH)])
      return 0
    lax.fori_loop(0, PERT // CH, zero_acc_copy, 0)

    def init_acc(j, val):
      acc_v[pl.ds(j * L, L)] = val
      return val
    lax.fori_loop(0, NPAD // L, init_acc, jnp.full((L,), -3e38, jnp.float32))

    src_prime()

    def phase_a(i, _):
      slot = src_wait(i)
      for j in range(JV):
        off = pl.ds(j * L, L)
        src16 = srcb_v[slot, off]
        dst16 = dst_t[i, off]
        s = (plsc.load_gather(asrc_v, [src16])
             + plsc.load_gather(adst_v, [dst16])
             + sx_t[i, off])
        s = jnp.where(s > 0, s, ALPHA * s)
        sx_t[i, off] = s
        dsts, ss = plsc.sort_key_val(dst16, s)
        m = ss
        for k in (1, 2, 4, 8):
          idx = jnp.minimum(_iota16() + k, L - 1)
          cand = jnp.take_along_axis(m, idx, axis=0, mode="promise_in_bounds")
          same = dsts == jnp.take_along_axis(dsts, idx, axis=0,
                                             mode="promise_in_bounds")
          m = jnp.where(same, jnp.maximum(m, cand), m)
        first, _ = _run_masks(dsts)
        cur = plsc.load_gather(acc_v, [dsts])
        plsc.store_scatter(acc_v, [dsts], jnp.maximum(cur, m), mask=first)
      return 0
    lax.fori_loop(0, NCHUNK, phase_a, 0)

    def repair_sweep(i, ch):
      for j in range(JV):
        off = pl.ds(j * L, L)
        dst16 = dst_t[i, off]
        s = sx_t[i, off]
        cur = plsc.load_gather(acc_v, [dst16])
        need = s > cur
        plsc.store_scatter(acc_v, [dst16], jnp.maximum(cur, s), mask=need)
        ch = ch | jnp.any(need)
      return ch

    lax.while_loop(
        lambda c: c,
        lambda c: lax.fori_loop(0, NCHUNK, repair_sweep, jnp.bool_(False)),
        jnp.bool_(True))

    def combine(op_is_max):
      plsc.subcore_barrier()

      def round_body(r, _):
        pltpu.sync_copy(acc_v.at[pl.ds(r * CW, CW)],
                        comb_s.at[pl.ds(sid * CW, CW)])
        plsc.subcore_barrier()
        for t in range(NS):
          pltpu.sync_copy(comb_s.at[pl.ds(t * CW + sid * SUB, SUB)],
                          cstage_v.at[pl.ds(t * SUB, SUB)])
        for j in range(SUB // L):
          v = cstage_v[pl.ds(j * L, L)]
          for t in range(1, NS):
            w = cstage_v[pl.ds(t * SUB + j * L, L)]
            v = jnp.maximum(v, w) if op_is_max else v + w
          redtmp_v[pl.ds(j * L, L)] = v
        pltpu.sync_copy(redtmp_v, red_s.at[pl.ds(r * CW + sid * SUB, SUB)])
        plsc.subcore_barrier()
        return 0
      lax.fori_loop(0, NPAD // CW, round_body, 0)
      pltpu.sync_copy(red_s, comb_v)

    combine(op_is_max=True)

    lax.fori_loop(0, NPAD // L, init_acc, jnp.zeros((L,), jnp.float32))

    def phase_b(i, _):
      for j in range(JV):
        off = pl.ds(j * L, L)
        dst16 = dst_t[i, off]
        m16 = plsc.load_gather(comb_v, [dst16])
        ex = jnp.exp(jnp.minimum(sx_t[i, off] - m16, 80.0))
        sx_t[i, off] = ex
        dsts, exs = plsc.sort_key_val(dst16, ex)
        c = plsc.cumsum(exs)
        prev = jnp.where(_iota16() == 0, 0.0, _shift_right1(c))
        first, last = _run_masks(dsts)
        base = plsc.cummax(jnp.where(first, prev, 0.0))
        run_sum = c - base
        plsc.addupdate_scatter(acc_v, [dsts], run_sum, mask=last)
      return 0
    lax.fori_loop(0, NCHUNK, phase_b, 0)

    combine(op_is_max=False)

    def c_pipeline(plane, with_stats):
      ssem = (sem2, sem3)

      pltpu.async_copy(src_hbm.at[sid, 0], srcb_v.at[0], sem2)
      pltpu.async_copy(src_hbm.at[sid, 1], srcb_v.at[1], sem3)
      pltpu.make_async_copy(src_hbm.at[sid, 0], srcb_v.at[0], sem2).wait()
      pltpu.async_copy(hq.at[plane].at[srcb_v.at[0]], rowb_v.at[0], sem)

      def body2(i2, carry):
        sa, sa2 = carry
        for par in (0, 1):
          oth = 1 - par
          i = 2 * i2 + par
          if with_stats:
            for j in range(JV):
              off = pl.ds(j * L, L)
              dst16 = dst_t[i, off]
              den16 = plsc.load_gather(comb_v, [dst16])
              a = sx_t[i, off] / (den16 + 1e-9)
              sx_t[i, off] = a
              sa = sa + a
              sa2 = sa2 + a * a
            eoff = sid * EPT + i * CH
            pltpu.sync_copy(e_tab.at[hh, pl.ds(eoff, CH)], erow_v)

          pltpu.make_async_copy(hq.at[plane].at[srcb_v.at[par]],
                                rowb_v.at[par], sem).wait()

          @pl.when(i + 2 < NCHUNK)
          def _():
            pltpu.async_copy(src_hbm.at[sid, i + 2], srcb_v.at[par],
                             ssem[par])

          @pl.when(i >= 1)
          def _():
            pltpu.make_async_copy(rowb_v.at[oth],
                                  outh_s.at[dst_t.at[i - 1]], sem4).wait()

          @pl.when(i + 1 < NCHUNK)
          def _():
            pltpu.make_async_copy(src_hbm.at[sid, i + 1], srcb_v.at[oth],
                                  ssem[oth]).wait()
            pltpu.async_copy(hq.at[plane].at[srcb_v.at[oth]],
                             rowb_v.at[oth], sem)

          def scale_16rows(j, _):
            a16 = sx_t[i, pl.ds(j * L, L)]
            r0 = j * L
            for r in range(L):
              ar = lax.broadcast(a16[r], (L,))
              row = r0 + r
              for k in range(DQ // L):
                o = pl.ds(k * L, L)
                rowb_v[par, row, o] = rowb_v[par, row, o] * ar
              if with_stats:
                erow_v[row, :] = erow_v[row, :] * ar
            return 0
          lax.fori_loop(0, JV, scale_16rows, 0)

          pltpu.async_copy(rowb_v.at[par], outh_s.at[dst_t.at[i]], sem4,
                           add=True)
          if with_stats:
            pltpu.sync_copy(erow_v, oute_s.at[dst_t.at[i]], add=True)
        return (sa, sa2)

      out = lax.fori_loop(0, NCHUNK // 2, body2,
                          (jnp.zeros((L,), jnp.float32),
                           jnp.zeros((L,), jnp.float32)))
      pltpu.make_async_copy(rowb_v.at[1],
                            outh_s.at[dst_t.at[NCHUNK - 1]], sem4).wait()
      return out

    sa, sa2 = c_pipeline(hh * QP, with_stats=True)

    it = _iota16()
    stat_v[:] = jnp.where(it == 0, jnp.sum(sa),
                          jnp.where(it == 1, jnp.sum(sa2), 0.0))
    pltpu.sync_copy(stat_v, stats_s.at[pl.ds(sid * L, L)])
    plsc.subcore_barrier()

    @pl.when(sid == 0)
    def _():
      pltpu.sync_copy(stats_s, statall_v)
      r = statall_v[pl.ds(0, L)]
      for t in range(1, NS):
        r = r + statall_v[pl.ds(t * L, L)]
      stat_v[:] = r
      pltpu.sync_copy(stat_v, stats.at[hh])

    base = sid * PERT

    pltpu.sync_copy(oute_s.at[pl.ds(base, PERT)],
                    oeq.at[hh, pl.ds(base, PERT)])

    def h_pass(q, _):
      plsc.subcore_barrier()
      pltpu.sync_copy(outh_s.at[pl.ds(base, PERT)],
                      ohq.at[hh * QP + q - 1, pl.ds(base, PERT)])
      lax.fori_loop(0, CH, zero_rowb, 0)
      lax.fori_loop(0, PERT // CH, zero_outh, 0)
      plsc.subcore_barrier()
      c_pipeline(hh * QP + q, with_stats=False)
      return 0
    lax.fori_loop(1, QP, h_pass, 0)

    plsc.subcore_barrier()
    pltpu.sync_copy(outh_s.at[pl.ds(base, PERT)],
                    ohq.at[hh * QP + QP - 1, pl.ds(base, PERT)])
    plsc.subcore_barrier()
    return 0

  lax.fori_loop(0, H // NC, process_head, 0)


def _make_sc_call():
  mesh = plsc.VectorSubcoreMesh(core_axis_name="c", subcore_axis_name="s",
                                num_cores=NC, num_subcores=NS)
  out_type = (
      jax.ShapeDtypeStruct((H * QP, NPAD, DQ), jnp.float32),
      jax.ShapeDtypeStruct((H, NPAD, E_OUT), jnp.float32),
      jax.ShapeDtypeStruct((H, L), jnp.float32),
  )
  scratch = [
      pltpu.VMEM((2, CH), jnp.int32),
      pltpu.VMEM((NCHUNK, CH), jnp.int32),
      pltpu.VMEM((NCHUNK, CH), jnp.float32),
      pltpu.VMEM((NPAD,), jnp.float32),
      pltpu.VMEM((NPAD,), jnp.float32),
      pltpu.VMEM((NPAD,), jnp.float32),
      pltpu.VMEM((NPAD,), jnp.float32),
      pltpu.VMEM((NS * SUB,), jnp.float32),
      pltpu.VMEM((SUB,), jnp.float32),
      pltpu.VMEM((2, CH, DQ), jnp.float32),
      pltpu.VMEM((CH, E_OUT), jnp.float32),
      pltpu.VMEM((L,), jnp.float32),
      pltpu.VMEM((NS * L,), jnp.float32),
      pltpu.VMEM_SHARED((NPAD, DQ), jnp.float32),
      pltpu.VMEM_SHARED((NPAD, E_OUT), jnp.float32),
      pltpu.VMEM_SHARED((NS * CW,), jnp.float32),
      pltpu.VMEM_SHARED((NPAD,), jnp.float32),
      pltpu.VMEM_SHARED((NS * L,), jnp.float32),
      pltpu.SemaphoreType.DMA,
      pltpu.SemaphoreType.DMA,
      pltpu.SemaphoreType.DMA,
      pltpu.SemaphoreType.DMA,
  ]
  return pl.kernel(_sc_kernel_body, out_type=out_type, mesh=mesh,
                   scratch_types=scratch,
                   compiler_params=pltpu.CompilerParams(
                       needs_layout_passes=False,
                       use_tc_tiling_on_sc=False))


def _tc_node_proj(node_fts, wn_all, a_alpha):
  nb = 1000
  grid = (N // nb,)

  def body(x_ref, w_ref, a_ref, *out_refs):
    h = jnp.dot(x_ref[...], w_ref[...], preferred_element_type=jnp.float32)
    for t in range(H):
      for q in range(QP):
        c0 = t * D_OUT + q * DQ
        out_refs[t * QP + q][...] = h[:, c0:c0 + DQ]
    out_refs[H * QP][...] = jnp.dot(h, a_ref[...],
                                    preferred_element_type=jnp.float32)

  return pl.pallas_call(
      body,
      grid=grid,
      in_specs=[
          pl.BlockSpec((nb, D_IN), lambda i: (i, 0)),
          pl.BlockSpec((D_IN, H * D_OUT), lambda i: (0, 0)),
          pl.BlockSpec((H * D_OUT, 2 * H), lambda i: (0, 0)),
      ],
      out_specs=[pl.BlockSpec((nb, DQ), lambda i: (i, 0))
                 for _ in range(H * QP)]
      + [pl.BlockSpec((nb, 2 * H), lambda i: (i, 0))],
      out_shape=[jax.ShapeDtypeStruct((N, DQ), jnp.float32)
                 for _ in range(H * QP)]
      + [jax.ShapeDtypeStruct((N, 2 * H), jnp.float32)],
  )(node_fts, wn_all, a_alpha)


def _tc_edge_proj(edge_fts, we_all, a_e):
  eb = 4000
  grid = (E // eb,)

  def body(x_ref, w_ref, a_ref, e_ref, s0_ref):
    e = jnp.dot(x_ref[...], w_ref[...], preferred_element_type=jnp.float32)
    for t in range(H):
      e_ref[t, ...] = e[:, t * E_OUT:(t + 1) * E_OUT]
    s0_ref[...] = jnp.dot(e, a_ref[...], preferred_element_type=jnp.float32)

  return pl.pallas_call(
      body,
      grid=grid,
      in_specs=[
          pl.BlockSpec((eb, E_IN), lambda i: (i, 0)),
          pl.BlockSpec((E_IN, H * E_OUT), lambda i: (0, 0)),
          pl.BlockSpec((H * E_OUT, H), lambda i: (0, 0)),
      ],
      out_specs=[pl.BlockSpec((H, eb, E_OUT), lambda i: (0, i, 0)),
                 pl.BlockSpec((eb, H), lambda i: (i, 0))],
      out_shape=[jax.ShapeDtypeStruct((H, E, E_OUT), jnp.float32),
                 jax.ShapeDtypeStruct((E, H), jnp.float32)],
  )(edge_fts, we_all, a_e)


def _tc_finalize(ohq, oeq, stats):
  nb = 1000
  grid = (N // nb,)
  dcat = H * (D_OUT + E_OUT)

  def body(ohq_ref, oeq_ref, st_ref, out_ref):
    st = st_ref[...]
    sum_a = st[:, 0:1]
    sum_a2 = st[:, 1:2]
    ef = jnp.float32(E)
    mean = sum_a / ef
    var = sum_a2 / ef - mean * mean
    v = jnp.exp(jnp.clip(var, -2.0, 2.0))
    v = v / jnp.sum(v)
    pieces = []
    for t in range(H):
      sc = v[t, 0]
      for q in range(QP):
        pieces.append(ohq_ref[t * QP + q, ...] * sc)
      pieces.append(oeq_ref[t, ...] * sc)
    out_ref[...] = jnp.concatenate(pieces, axis=1)

  return pl.pallas_call(
      body,
      grid=grid,
      in_specs=[
          pl.BlockSpec((H * QP, nb, DQ), lambda i: (0, i, 0)),
          pl.BlockSpec((H, nb, E_OUT), lambda i: (0, i, 0)),
          pl.BlockSpec((H, L), lambda i: (0, 0)),
      ],
      out_specs=pl.BlockSpec((nb, dcat), lambda i: (i, 0)),
      out_shape=jax.ShapeDtypeStruct((N, dcat), jnp.float32),
  )(ohq, oeq, stats)


@jax.jit
def kernel(node_fts, edge_fts, edges, W_node, W_edge, attn_a):
  src = edges[:, 0].astype(jnp.int32).reshape(NS, NCHUNK, CH)
  dst = edges[:, 1].astype(jnp.int32).reshape(NS, NCHUNK, CH)

  wn_all = jnp.transpose(W_node, (1, 0, 2)).reshape(D_IN, H * D_OUT)
  we_all = jnp.transpose(W_edge, (1, 0, 2)).reshape(E_IN, H * E_OUT)

  a1 = attn_a[:, :D_OUT]
  a2 = attn_a[:, D_OUT:2 * D_OUT]
  a3 = attn_a[:, 2 * D_OUT:2 * D_OUT + E_OUT]
  eye = jnp.eye(H, dtype=jnp.float32)
  a_src_m = (a1[:, :, None] * eye[:, None, :]).reshape(H * D_OUT, H)
  a_dst_m = (a2[:, :, None] * eye[:, None, :]).reshape(H * D_OUT, H)
  a_alpha = jnp.concatenate([a_src_m, a_dst_m], axis=1)
  a_e = (a3[:, :, None] * eye[:, None, :]).reshape(H * E_OUT, H)

  *hqs, alpha = _tc_node_proj(node_fts, wn_all, a_alpha)
  hq = jnp.stack(hqs)
  e_tab, s0 = _tc_edge_proj(edge_fts, we_all, a_e)

  alphaT = alpha.T
  pad = NPAD - N
  as_tab = jnp.pad(alphaT[:H], ((0, 0), (0, pad)))
  ad_tab = jnp.pad(alphaT[H:], ((0, 0), (0, pad)))
  s0q = s0.T.reshape(H, NS, NCHUNK, CH)

  ohq, oeq, stats = _make_sc_call()(src, dst, hq, e_tab, s0q, as_tab, ad_tab)
  return _tc_finalize(ohq, oeq, stats)


# speedup vs baseline: 8.4696x; 1.0549x over previous
import jax
import jax.numpy as jnp
from jax import lax
from jax.experimental import pallas as pl
from jax.experimental.pallas import tpu as pltpu
from jax.experimental.pallas import tpu_sc as plsc

N = 10000
E = 320000
D_IN = 128
D_OUT = 128
E_IN = 16
E_OUT = 16
H = 4
ALPHA = 0.2

NC = 2
NS = 16
L = 16

NPAD = 10240
PERT = NPAD // NS
EPT = E // NS
CH = 80
NCHUNK = EPT // CH
JV = CH // L
CW = 512
SUB = CW // NS
QP = 4
DQ = D_OUT // QP


def _iota16():
  return lax.iota(jnp.int32, L)


def _shift_left(v, k):
  idx = jnp.minimum(_iota16() + k, L - 1)
  return jnp.take_along_axis(v, idx, axis=0, mode="promise_in_bounds")


def _shift_right1(v):
  idx = jnp.maximum(_iota16() - 1, 0)
  return jnp.take_along_axis(v, idx, axis=0, mode="promise_in_bounds")


def _run_masks(dsts):
  it = _iota16()
  first = (it == 0) | (dsts != _shift_right1(dsts))
  last = (it == L - 1) | (dsts != _shift_left(dsts, 1))
  return first, last


def _sc_kernel_body(src_hbm, dst_hbm, hq, e_tab, s0q, as_tab, ad_tab,
                    ohq, oeq, stats,
                    srcb_v, dst_t, sx_t,
                    asrc_v, adst_v, acc_v, comb_v,
                    cstage_v, redtmp_v, rowb_v, erow_v, stat_v, statall_v,
                    outh_s, oute_s, comb_s, red_s, stats_s,
                    sem, sem2, sem3, sem4):
  sid = lax.axis_index("s")
  cid = lax.axis_index("c")

  zero16 = jnp.zeros((L,), jnp.float32)

  pltpu.sync_copy(dst_hbm.at[sid], dst_t)

  def src_wait(i):
    slot = lax.rem(i, 2)
    pltpu.make_async_copy(src_hbm.at[sid, i], srcb_v.at[slot], sem2).wait()

    @pl.when(i + 1 < NCHUNK)
    def _():
      pltpu.async_copy(src_hbm.at[sid, i + 1],
                       srcb_v.at[lax.rem(i + 1, 2)], sem2)
    return slot

  def src_prime():
    pltpu.async_copy(src_hbm.at[sid, 0], srcb_v.at[0], sem2)

  def zero_rowb(r, _):
    for k in range(DQ // L):
      rowb_v[0, r, pl.ds(k * L, L)] = zero16
    erow_v[r, :] = zero16
    return 0

  def zero_outh(j, _):
    pltpu.sync_copy(rowb_v.at[0], outh_s.at[pl.ds(sid * PERT + j * CH, CH)])
    return 0

  def process_head(t, _):
    hh = cid * 2 + t
    pltpu.sync_copy(s0q.at[hh, sid], sx_t)
    pltpu.sync_copy(as_tab.at[hh], asrc_v)
    pltpu.sync_copy(ad_tab.at[hh], adst_v)

    lax.fori_loop(0, CH, zero_rowb, 0)

    def zero_acc_copy(j, _):
      base = sid * PERT + j * CH
      pltpu.sync_copy(rowb_v.at[0], outh_s.at[pl.ds(base, CH)])
      pltpu.sync_copy(erow_v, oute_s.at[pl.ds(base, CH)])
      return 0
    lax.fori_loop(0, PERT // CH, zero_acc_copy, 0)

    def init_acc(j, val):
      acc_v[pl.ds(j * L, L)] = val
      return val
    lax.fori_loop(0, NPAD // L, init_acc, jnp.full((L,), -3e38, jnp.float32))

    src_prime()

    def phase_a(i, _):
      slot = src_wait(i)
      for j in range(JV):
        off = pl.ds(j * L, L)
        src16 = srcb_v[slot, off]
        dst16 = dst_t[i, off]
        s = (plsc.load_gather(asrc_v, [src16])
             + plsc.load_gather(adst_v, [dst16])
             + sx_t[i, off])
        s = jnp.where(s > 0, s, ALPHA * s)
        sx_t[i, off] = s
        dsts, ss = plsc.sort_key_val(dst16, s)
        m = ss
        for k in (1, 2, 4, 8):
          idx = jnp.minimum(_iota16() + k, L - 1)
          cand = jnp.take_along_axis(m, idx, axis=0, mode="promise_in_bounds")
          same = dsts == jnp.take_along_axis(dsts, idx, axis=0,
                                             mode="promise_in_bounds")
          m = jnp.where(same, jnp.maximum(m, cand), m)
        first, _ = _run_masks(dsts)
        cur = plsc.load_gather(acc_v, [dsts])
        plsc.store_scatter(acc_v, [dsts], jnp.maximum(cur, m), mask=first)
      return 0
    lax.fori_loop(0, NCHUNK, phase_a, 0)

    def repair_sweep(i, ch):
      for j in range(JV):
        off = pl.ds(j * L, L)
        dst16 = dst_t[i, off]
        s = sx_t[i, off]
        cur = plsc.load_gather(acc_v, [dst16])
        need = s > cur
        plsc.store_scatter(acc_v, [dst16], jnp.maximum(cur, s), mask=need)
        ch = ch | jnp.any(need)
      return ch

    lax.while_loop(
        lambda c: c,
        lambda c: lax.fori_loop(0, NCHUNK, repair_sweep, jnp.bool_(False)),
        jnp.bool_(True))

    def combine(op_is_max):
      plsc.subcore_barrier()

      def round_body(r, _):
        pltpu.sync_copy(acc_v.at[pl.ds(r * CW, CW)],
                        comb_s.at[pl.ds(sid * CW, CW)])
        plsc.subcore_barrier()
        for t in range(NS):
          pltpu.sync_copy(comb_s.at[pl.ds(t * CW + sid * SUB, SUB)],
                          cstage_v.at[pl.ds(t * SUB, SUB)])
        for j in range(SUB // L):
          v = cstage_v[pl.ds(j * L, L)]
          for t in range(1, NS):
            w = cstage_v[pl.ds(t * SUB + j * L, L)]
            v = jnp.maximum(v, w) if op_is_max else v + w
          redtmp_v[pl.ds(j * L, L)] = v
        pltpu.sync_copy(redtmp_v, red_s.at[pl.ds(r * CW + sid * SUB, SUB)])
        plsc.subcore_barrier()
        return 0
      lax.fori_loop(0, NPAD // CW, round_body, 0)
      pltpu.sync_copy(red_s, comb_v)

    combine(op_is_max=True)

    lax.fori_loop(0, NPAD // L, init_acc, jnp.zeros((L,), jnp.float32))

    def phase_b(i, _):
      for j in range(JV):
        off = pl.ds(j * L, L)
        dst16 = dst_t[i, off]
        m16 = plsc.load_gather(comb_v, [dst16])
        ex = jnp.exp(jnp.minimum(sx_t[i, off] - m16, 80.0))
        sx_t[i, off] = ex
        dsts, exs = plsc.sort_key_val(dst16, ex)
        c = plsc.cumsum(exs)
        prev = jnp.where(_iota16() == 0, 0.0, _shift_right1(c))
        first, last = _run_masks(dsts)
        base = plsc.cummax(jnp.where(first, prev, 0.0))
        run_sum = c - base
        plsc.addupdate_scatter(acc_v, [dsts], run_sum, mask=last)
      return 0
    lax.fori_loop(0, NCHUNK, phase_b, 0)

    combine(op_is_max=False)

    def c_pipeline(plane, with_stats):
      ssem = (sem2, sem3)

      pltpu.async_copy(src_hbm.at[sid, 0], srcb_v.at[0], sem2)
      pltpu.async_copy(src_hbm.at[sid, 1], srcb_v.at[1], sem3)
      pltpu.make_async_copy(src_hbm.at[sid, 0], srcb_v.at[0], sem2).wait()
      pltpu.async_copy(hq.at[plane].at[srcb_v.at[0]], rowb_v.at[0], sem)

      def body2(i2, carry):
        sa, sa2 = carry
        for par in (0, 1):
          oth = 1 - par
          i = 2 * i2 + par
          if with_stats:
            for j in range(JV):
              off = pl.ds(j * L, L)
              dst16 = dst_t[i, off]
              den16 = plsc.load_gather(comb_v, [dst16])
              a = sx_t[i, off] / (den16 + 1e-9)
              sx_t[i, off] = a
              sa = sa + a
              sa2 = sa2 + a * a
            eoff = sid * EPT + i * CH
            pltpu.sync_copy(e_tab.at[hh, pl.ds(eoff, CH)], erow_v)

          pltpu.make_async_copy(hq.at[plane].at[srcb_v.at[par]],
                                rowb_v.at[par], sem).wait()

          @pl.when(i + 2 < NCHUNK)
          def _():
            pltpu.async_copy(src_hbm.at[sid, i + 2], srcb_v.at[par],
                             ssem[par])

          @pl.when(i >= 1)
          def _():
            pltpu.make_async_copy(rowb_v.at[oth],
                                  outh_s.at[dst_t.at[i - 1]], sem4).wait()

          @pl.when(i + 1 < NCHUNK)
          def _():
            pltpu.make_async_copy(src_hbm.at[sid, i + 1], srcb_v.at[oth],
                                  ssem[oth]).wait()
            pltpu.async_copy(hq.at[plane].at[srcb_v.at[oth]],
                             rowb_v.at[oth], sem)

          def scale_16rows(j, _):
            a16 = sx_t[i, pl.ds(j * L, L)]
            r0 = j * L
            for r in range(L):
              ar = lax.broadcast(a16[r], (L,))
              row = r0 + r
              for k in range(DQ // L):
                o = pl.ds(k * L, L)
                rowb_v[par, row, o] = rowb_v[par, row, o] * ar
              if with_stats:
                erow_v[row, :] = erow_v[row, :] * ar
            return 0
          lax.fori_loop(0, JV, scale_16rows, 0)

          pltpu.async_copy(rowb_v.at[par], outh_s.at[dst_t.at[i]], sem4,
                           add=True)
          if with_stats:
            pltpu.sync_copy(erow_v, oute_s.at[dst_t.at[i]], add=True)
        return (sa, sa2)

      out = lax.fori_loop(0, NCHUNK // 2, body2,
                          (jnp.zeros((L,), jnp.float32),
                           jnp.zeros((L,), jnp.float32)))
      pltpu.make_async_copy(rowb_v.at[1],
                            outh_s.at[dst_t.at[NCHUNK - 1]], sem4).wait()
      return out

    sa, sa2 = c_pipeline(hh * QP, with_stats=True)

    it = _iota16()
    stat_v[:] = jnp.where(it == 0, jnp.sum(sa),
                          jnp.where(it == 1, jnp.sum(sa2), 0.0))
    pltpu.sync_copy(stat_v, stats_s.at[pl.ds(sid * L, L)])
    plsc.subcore_barrier()

    @pl.when(sid == 0)
    def _():
      pltpu.sync_copy(stats_s, statall_v)
      r = statall_v[pl.ds(0, L)]
      for t in range(1, NS):
        r = r + statall_v[pl.ds(t * L, L)]
      stat_v[:] = r
      pltpu.sync_copy(stat_v, stats.at[hh])

    base = sid * PERT

    pltpu.sync_copy(oute_s.at[pl.ds(base, PERT)],
                    oeq.at[hh, pl.ds(base, PERT)])

    def h_pass(q, _):
      plsc.subcore_barrier()
      pltpu.sync_copy(outh_s.at[pl.ds(base, PERT)],
                      ohq.at[hh * QP + q - 1, pl.ds(base, PERT)])
      lax.fori_loop(0, CH, zero_rowb, 0)
      lax.fori_loop(0, PERT // CH, zero_outh, 0)
      plsc.subcore_barrier()
      c_pipeline(hh * QP + q, with_stats=False)
      return 0
    lax.fori_loop(1, QP, h_pass, 0)

    plsc.subcore_barrier()
    pltpu.sync_copy(outh_s.at[pl.ds(base, PERT)],
                    ohq.at[hh * QP + QP - 1, pl.ds(base, PERT)])
    plsc.subcore_barrier()
    return 0

  lax.fori_loop(0, H // NC, process_head, 0)


def _make_sc_call():
  mesh = plsc.VectorSubcoreMesh(core_axis_name="c", subcore_axis_name="s",
                                num_cores=NC, num_subcores=NS)
  out_type = (
      jax.ShapeDtypeStruct((H * QP, NPAD, DQ), jnp.float32),
      jax.ShapeDtypeStruct((H, NPAD, E_OUT), jnp.float32),
      jax.ShapeDtypeStruct((H, L), jnp.float32),
  )
  scratch = [
      pltpu.VMEM((2, CH), jnp.int32),
      pltpu.VMEM((NCHUNK, CH), jnp.int32),
      pltpu.VMEM((NCHUNK, CH), jnp.float32),
      pltpu.VMEM((NPAD,), jnp.float32),
      pltpu.VMEM((NPAD,), jnp.float32),
      pltpu.VMEM((NPAD,), jnp.float32),
      pltpu.VMEM((NPAD,), jnp.float32),
      pltpu.VMEM((NS * SUB,), jnp.float32),
      pltpu.VMEM((SUB,), jnp.float32),
      pltpu.VMEM((2, CH, DQ), jnp.float32),
      pltpu.VMEM((CH, E_OUT), jnp.float32),
      pltpu.VMEM((L,), jnp.float32),
      pltpu.VMEM((NS * L,), jnp.float32),
      pltpu.VMEM_SHARED((NPAD, DQ), jnp.float32),
      pltpu.VMEM_SHARED((NPAD, E_OUT), jnp.float32),
      pltpu.VMEM_SHARED((NS * CW,), jnp.float32),
      pltpu.VMEM_SHARED((NPAD,), jnp.float32),
      pltpu.VMEM_SHARED((NS * L,), jnp.float32),
      pltpu.SemaphoreType.DMA,
      pltpu.SemaphoreType.DMA,
      pltpu.SemaphoreType.DMA,
      pltpu.SemaphoreType.DMA,
  ]
  return pl.kernel(_sc_kernel_body, out_type=out_type, mesh=mesh,
                   scratch_types=scratch,
                   compiler_params=pltpu.CompilerParams(
                       needs_layout_passes=False,
                       use_tc_tiling_on_sc=False))


def _tc_node_proj(node_fts, wn_all, a_alpha):
  nb = 1000
  grid = (N // nb,)

  def body(x_ref, w_ref, a_ref, hq_ref, al_ref):
    h = jnp.dot(x_ref[...], w_ref[...], preferred_element_type=jnp.float32)
    for t in range(H):
      for q in range(QP):
        c0 = t * D_OUT + q * DQ
        hq_ref[t * QP + q, :, :] = h[:, c0:c0 + DQ]
    al_ref[0, :, :] = jnp.dot(h, a_ref[...],
                              preferred_element_type=jnp.float32)

  return pl.pallas_call(
      body,
      grid=grid,
      in_specs=[
          pl.BlockSpec((nb, D_IN), lambda i: (i, 0)),
          pl.BlockSpec((D_IN, H * D_OUT), lambda i: (0, 0)),
          pl.BlockSpec((H * D_OUT, 2 * H), lambda i: (0, 0)),
      ],
      out_specs=[pl.BlockSpec((H * QP, nb, DQ), lambda i: (0, i, 0)),
                 pl.BlockSpec((1, nb, 2 * H), lambda i: (i, 0, 0))],
      out_shape=[jax.ShapeDtypeStruct((H * QP, N, DQ), jnp.float32),
                 jax.ShapeDtypeStruct((N // nb, nb, 2 * H), jnp.float32)],
  )(node_fts, wn_all, a_alpha)


def _tc_edge_proj(edge_fts, we_all, a_e):
  eb = 4000
  grid = (E // eb,)

  def body(x_ref, w_ref, a_ref, e_ref, s0_ref):
    e = jnp.dot(x_ref[...], w_ref[...], preferred_element_type=jnp.float32)
    for t in range(H):
      e_ref[t, ...] = e[:, t * E_OUT:(t + 1) * E_OUT]
    s0_ref[0, :, :] = lax.dot_general(
        a_ref[...], e, (((0,), (1,)), ((), ())),
        preferred_element_type=jnp.float32)

  return pl.pallas_call(
      body,
      grid=grid,
      in_specs=[
          pl.BlockSpec((eb, E_IN), lambda i: (i, 0)),
          pl.BlockSpec((E_IN, H * E_OUT), lambda i: (0, 0)),
          pl.BlockSpec((H * E_OUT, H), lambda i: (0, 0)),
      ],
      out_specs=[pl.BlockSpec((H, eb, E_OUT), lambda i: (0, i, 0)),
                 pl.BlockSpec((1, H, eb), lambda i: (i, 0, 0))],
      out_shape=[jax.ShapeDtypeStruct((H, E, E_OUT), jnp.float32),
                 jax.ShapeDtypeStruct((E // eb, H, eb), jnp.float32)],
  )(edge_fts, we_all, a_e)


def _tc_finalize(ohq, oeq, stats):
  nb = 1000
  grid = (N // nb,)
  dcat = H * (D_OUT + E_OUT)

  def body(ohq_ref, oeq_ref, st_ref, out_ref):
    st = st_ref[...]
    sum_a = st[:, 0:1]
    sum_a2 = st[:, 1:2]
    ef = jnp.float32(E)
    mean = sum_a / ef
    var = sum_a2 / ef - mean * mean
    v = jnp.exp(jnp.clip(var, -2.0, 2.0))
    v = v / jnp.sum(v)
    pieces = []
    for t in range(H):
      sc = v[t, 0]
      for q in range(QP):
        pieces.append(ohq_ref[t * QP + q, ...] * sc)
      pieces.append(oeq_ref[t, ...] * sc)
    out_ref[...] = jnp.concatenate(pieces, axis=1)

  return pl.pallas_call(
      body,
      grid=grid,
      in_specs=[
          pl.BlockSpec((H * QP, nb, DQ), lambda i: (0, i, 0)),
          pl.BlockSpec((H, nb, E_OUT), lambda i: (0, i, 0)),
          pl.BlockSpec((H, L), lambda i: (0, 0)),
      ],
      out_specs=pl.BlockSpec((nb, dcat), lambda i: (i, 0)),
      out_shape=jax.ShapeDtypeStruct((N, dcat), jnp.float32),
  )(ohq, oeq, stats)


@jax.jit
def kernel(node_fts, edge_fts, edges, W_node, W_edge, attn_a):
  src = edges[:, 0].astype(jnp.int32).reshape(NS, NCHUNK, CH)
  dst = edges[:, 1].astype(jnp.int32).reshape(NS, NCHUNK, CH)

  wn_all = jnp.transpose(W_node, (1, 0, 2)).reshape(D_IN, H * D_OUT)
  we_all = jnp.transpose(W_edge, (1, 0, 2)).reshape(E_IN, H * E_OUT)

  a1 = attn_a[:, :D_OUT]
  a2 = attn_a[:, D_OUT:2 * D_OUT]
  a3 = attn_a[:, 2 * D_OUT:2 * D_OUT + E_OUT]
  eye = jnp.eye(H, dtype=jnp.float32)
  a_src_m = (a1[:, :, None] * eye[:, None, :]).reshape(H * D_OUT, H)
  a_dst_m = (a2[:, :, None] * eye[:, None, :]).reshape(H * D_OUT, H)
  a_alpha = jnp.concatenate([a_src_m, a_dst_m], axis=1)
  a_e = (a3[:, :, None] * eye[:, None, :]).reshape(H * E_OUT, H)

  hq, alpha3 = _tc_node_proj(node_fts, wn_all, a_alpha)
  e_tab, s03 = _tc_edge_proj(edge_fts, we_all, a_e)

  alphaT = alpha3.reshape(N, 2 * H).T
  pad = NPAD - N
  as_tab = jnp.pad(alphaT[:H], ((0, 0), (0, pad)))
  ad_tab = jnp.pad(alphaT[H:], ((0, 0), (0, pad)))
  s0q = jnp.transpose(s03, (1, 0, 2)).reshape(H, NS, NCHUNK, CH)

  ohq, oeq, stats = _make_sc_call()(src, dst, hq, e_tab, s0q, as_tab, ad_tab)
  return _tc_finalize(ohq, oeq, stats)
